# Initial kernel scaffold; baseline (speedup 1.0000x reference)
import jax
import jax.numpy as jnp
from jax.experimental import pallas as pl


def kernel(pos_ids_0, pos_ids_1, cat_ids_0, cat_ids_1, continuous_feature, pos_table_0, pos_table_1, cat_table_0, cat_table_1, W_cont, b_cont):
    raise NotImplementedError("write your pallas kernel here")


# speedup vs baseline: 3.5327x; 3.5327x over previous
import functools

import jax
import jax.numpy as jnp
from jax import lax
from jax.experimental import pallas as pl
from jax.experimental.pallas import tpu as pltpu
from jax.experimental.pallas import tpu_sc as plsc

_B, _S = 4096, 200
_N = _B * _S
_NC, _NS = 2, 16
_NW = _NC * _NS
_PER_W = _N // _NW
_C = 128
_CHUNKS = _PER_W // _C


def _cont_proj(x, w, b):
    tb = 8192

    def body(x_ref, w_ref, b_ref, o_ref):
        o_ref[...] = (
            jnp.dot(x_ref[...], w_ref[...], preferred_element_type=jnp.float32)
            + b_ref[...]
        )

    return pl.pallas_call(
        body,
        grid=(_N // tb,),
        in_specs=[
            pl.BlockSpec((tb, 16), lambda i: (i, 0)),
            pl.BlockSpec((16, 32), lambda i: (0, 0)),
            pl.BlockSpec((1, 32), lambda i: (0, 0)),
        ],
        out_specs=pl.BlockSpec((tb, 32), lambda i: (i, 0)),
        out_shape=jax.ShapeDtypeStruct((_N, 32), jnp.float32),
    )(x, w, b)


def _sc_assemble(pid0, pid1, cid0, cid1, ce, p0t, p1t, c0t, c1t):
    mesh = plsc.VectorSubcoreMesh(core_axis_name="c", subcore_axis_name="s")

    @functools.partial(
        pl.kernel,
        mesh=mesh,
        compiler_params=pltpu.CompilerParams(use_tc_tiling_on_sc=False),
        out_type=jax.ShapeDtypeStruct((_N, 128), jnp.float32),
        scratch_types=[
            pltpu.VMEM((200, 64), jnp.float32),
            pltpu.VMEM((200, 64), jnp.float32),
            pltpu.VMEM((_C,), jnp.int32),
            pltpu.VMEM((_C,), jnp.int32),
            pltpu.VMEM((_C,), jnp.int32),
            pltpu.VMEM((_C,), jnp.int32),
            pltpu.VMEM((_C, 64), jnp.float32),
            pltpu.VMEM((_C, 32), jnp.float32),
            pltpu.VMEM((_C, 32), jnp.float32),
            pltpu.VMEM((_C, 128), jnp.float32),
            pltpu.SemaphoreType.DMA,
            pltpu.SemaphoreType.DMA,
        ],
    )
    def k(pid0_h, pid1_h, cid0_h, cid1_h, ce_h, p0t_h, p1t_h, c0t_h, c1t_h,
          out_h, p0v, p1v, pi0, pi1, ci0, ci1, c0b, c1b, ceb, ob,
          s0, s1):
        wid = lax.axis_index("s") * _NC + lax.axis_index("c")
        base = wid * _PER_W
        pltpu.sync_copy(p0t_h, p0v)
        pltpu.sync_copy(p1t_h, p1v)

        def chunk(g, carry):
            off = pl.multiple_of(base + g * _C, _C)
            pltpu.sync_copy(pid0_h.at[pl.ds(off, _C)], pi0)
            pltpu.sync_copy(pid1_h.at[pl.ds(off, _C)], pi1)
            pltpu.sync_copy(cid0_h.at[pl.ds(off, _C)], ci0)
            pltpu.sync_copy(cid1_h.at[pl.ds(off, _C)], ci1)
            cp0 = pltpu.async_copy(c0t_h.at[ci0], c0b, s0)
            cp1 = pltpu.async_copy(c1t_h.at[ci1], c1b, s1)
            pltpu.sync_copy(ce_h.at[pl.ds(off, _C)], ceb)
            cp0.wait()
            cp1.wait()

            def grpbody(g2, c2):
                t0 = pl.multiple_of(g2 * 16, 16)
                i0vec = pi0[pl.ds(t0, 16)]
                i1vec = pi1[pl.ds(t0, 16)]
                for l in range(16):
                    t = t0 + l
                    id0 = i0vec[l]
                    id1 = i1vec[l]
                    for v in range(4):
                        ob[t, pl.ds(16 * v, 16)] = (
                            c0b[t, pl.ds(16 * v, 16)]
                            + p0v[id0, pl.ds(16 * v, 16)])
                    for v in range(2):
                        ob[t, pl.ds(64 + 16 * v, 16)] = (
                            c1b[t, pl.ds(16 * v, 16)]
                            + p1v[id1, pl.ds(16 * v, 16)])
                    for v in range(2):
                        ob[t, pl.ds(96 + 16 * v, 16)] = (
                            ceb[t, pl.ds(16 * v, 16)]
                            + p1v[id1, pl.ds(32 + 16 * v, 16)])
                return c2

            lax.fori_loop(0, _C // 16, grpbody, 0)
            pltpu.sync_copy(ob, out_h.at[pl.ds(off, _C)])
            return carry

        lax.fori_loop(0, _CHUNKS, chunk, 0)

    return k(pid0, pid1, cid0, cid1, ce, p0t, p1t, c0t, c1t)


def kernel(pos_ids_0, pos_ids_1, cat_ids_0, cat_ids_1, continuous_feature,
           pos_table_0, pos_table_1, cat_table_0, cat_table_1, W_cont, b_cont):
    pid0 = pos_ids_0.reshape(_N).astype(jnp.int32)
    pid1 = pos_ids_1.reshape(_N).astype(jnp.int32)
    cid0 = cat_ids_0.reshape(_N).astype(jnp.int32)
    cid1 = cat_ids_1.reshape(_N).astype(jnp.int32)
    ce = _cont_proj(continuous_feature.reshape(_N, 16), W_cont,
                    b_cont.reshape(1, 32))
    out = _sc_assemble(pid0, pid1, cid0, cid1, ce,
                       pos_table_0, pos_table_1, cat_table_0, cat_table_1)
    return out.reshape(_B, _S, 128)


# speedup vs baseline: 5.1072x; 1.4457x over previous
import functools

import jax
import jax.numpy as jnp
from jax import lax
from jax.experimental import pallas as pl
from jax.experimental.pallas import tpu as pltpu
from jax.experimental.pallas import tpu_sc as plsc

_B, _S = 4096, 200
_N = _B * _S
_NC, _NS = 2, 16
_NW = _NC * _NS
_PER_W = _N // _NW
_C = 256
_CHUNKS = _PER_W // _C
_NBLK = _N // 128
_BPW = _PER_W // 128


def _cont_proj(x, w, b):
    tb = 8192

    def body(x_ref, w_ref, b_ref, o_ref):
        o_ref[...] = (
            jnp.dot(x_ref[...], w_ref[...], preferred_element_type=jnp.float32)
            + b_ref[...]
        )

    return pl.pallas_call(
        body,
        grid=(_N // tb,),
        in_specs=[
            pl.BlockSpec((tb, 16), lambda i: (i, 0)),
            pl.BlockSpec((16, 32), lambda i: (0, 0)),
            pl.BlockSpec((1, 32), lambda i: (0, 0)),
        ],
        out_specs=pl.BlockSpec((tb, 32), lambda i: (i, 0)),
        out_shape=jax.ShapeDtypeStruct((_N, 32), jnp.float32),
    )(x, w, b)


def _sc_assemble(pids, cids, ce, p0t, p1t, c0t, c1t):
    mesh = plsc.VectorSubcoreMesh(core_axis_name="c", subcore_axis_name="s")

    @functools.partial(
        pl.kernel,
        mesh=mesh,
        compiler_params=pltpu.CompilerParams(use_tc_tiling_on_sc=False),
        out_type=jax.ShapeDtypeStruct((_N, 128), jnp.float32),
        scratch_types=[
            pltpu.VMEM((200, 64), jnp.float32),
            pltpu.VMEM((200, 64), jnp.float32),
            [pltpu.VMEM((2, 2, 128), jnp.int32)] * 4,
            [pltpu.VMEM((2, 2, 128), jnp.int32)] * 4,
            [pltpu.VMEM((_C, 64), jnp.float32)] * 2,
            [pltpu.VMEM((_C, 32), jnp.float32)] * 2,
            [pltpu.VMEM((_C, 32), jnp.float32)] * 2,
            [pltpu.SemaphoreType.DMA] * 4,
            [pltpu.SemaphoreType.DMA] * 2,
            [pltpu.SemaphoreType.DMA] * 2,
        ],
    )
    def k(pids_h, cids_h, ce_h, p0t_h, p1t_h, c0t_h, c1t_h,
          out_h, p0v, p1v, pb, cb, c0b, c1b, ceb, semi, semb, semc):
        wid = lax.axis_index("s") * _NC + lax.axis_index("c")
        base = wid * _PER_W
        bbase = wid * _BPW
        pltpu.sync_copy(p0t_h, p0v)
        pltpu.sync_copy(p1t_h, p1v)

        def issue_a(c, s4):
            blk = bbase + 2 * c
            pltpu.async_copy(pids_h.at[:, pl.ds(blk, 2), :], pb[s4], semi[s4])
            pltpu.async_copy(cids_h.at[:, pl.ds(blk, 2), :], cb[s4], semi[s4])

        def wait_a(s4):
            pltpu.make_async_copy(
                pids_h.at[:, pl.ds(0, 2), :], pb[s4], semi[s4]).wait()
            pltpu.make_async_copy(
                cids_h.at[:, pl.ds(0, 2), :], cb[s4], semi[s4]).wait()

        def wait_c(s2):
            pltpu.make_async_copy(
                c0b[s2], out_h.at[pl.ds(0, _C), pl.ds(0, 64)], semc[s2]).wait()
            pltpu.make_async_copy(
                c1b[s2], out_h.at[pl.ds(0, _C), pl.ds(64, 32)],
                semc[s2]).wait()
            pltpu.make_async_copy(
                ceb[s2], out_h.at[pl.ds(0, _C), pl.ds(96, 32)],
                semc[s2]).wait()

        def issue_b(c, s4, s2, *, guard_out):
            if guard_out:
                @pl.when(c >= 2)
                def _():
                    wait_c(s2)
            wait_a(s4)
            for hb in range(2):
                pltpu.async_copy(
                    c0t_h.at[cb[s4].at[0, hb]],
                    c0b[s2].at[pl.ds(hb * 128, 128)], semb[s2])
                pltpu.async_copy(
                    c1t_h.at[cb[s4].at[1, hb]],
                    c1b[s2].at[pl.ds(hb * 128, 128)], semb[s2])
            off = base + c * _C
            pltpu.async_copy(ce_h.at[pl.ds(off, _C)], ceb[s2], semb[s2])

        def wait_b(s2):
            for hb in range(2):
                pltpu.make_async_copy(
                    c0t_h.at[cb[0].at[0, hb]],
                    c0b[s2].at[pl.ds(hb * 128, 128)], semb[s2]).wait()
                pltpu.make_async_copy(
                    c1t_h.at[cb[0].at[1, hb]],
                    c1b[s2].at[pl.ds(hb * 128, 128)], semb[s2]).wait()
            pltpu.make_async_copy(
                ce_h.at[pl.ds(0, _C)], ceb[s2], semb[s2]).wait()

        def compute(c, s4, s2):
            wait_b(s2)
            for hb in range(2):
                def grp(g2, c2):
                    l0 = pl.multiple_of(g2 * 16, 16)
                    i0vec = pb[s4][0, hb, pl.ds(l0, 16)]
                    i1vec = pb[s4][1, hb, pl.ds(l0, 16)]
                    for l in range(16):
                        t = hb * 128 + l0 + l
                        id0 = i0vec[l]
                        id1 = i1vec[l]
                        for v in range(4):
                            plsc.addupdate(
                                c0b[s2].at[t, pl.ds(16 * v, 16)],
                                p0v[id0, pl.ds(16 * v, 16)])
                        for v in range(2):
                            plsc.addupdate(
                                c1b[s2].at[t, pl.ds(16 * v, 16)],
                                p1v[id1, pl.ds(16 * v, 16)])
                        for v in range(2):
                            plsc.addupdate(
                                ceb[s2].at[t, pl.ds(16 * v, 16)],
                                p1v[id1, pl.ds(32 + 16 * v, 16)])
                    return c2

                lax.fori_loop(0, 8, grp, 0)
            off = base + c * _C
            pltpu.async_copy(
                c0b[s2], out_h.at[pl.ds(off, _C), pl.ds(0, 64)], semc[s2])
            pltpu.async_copy(
                c1b[s2], out_h.at[pl.ds(off, _C), pl.ds(64, 32)], semc[s2])
            pltpu.async_copy(
                ceb[s2], out_h.at[pl.ds(off, _C), pl.ds(96, 32)], semc[s2])

        issue_a(0, 0)
        issue_a(1, 1)
        issue_b(0, 0, 0, guard_out=False)

        def step(i, carry):
            for kk in range(4):
                c = 4 * i + kk

                @pl.when(c + 2 < _CHUNKS)
                def _():
                    issue_a(c + 2, (kk + 2) % 4)

                @pl.when(c + 1 < _CHUNKS)
                def _():
                    issue_b(c + 1, (kk + 1) % 4, (kk + 1) % 2, guard_out=True)

                compute(c, kk, kk % 2)
            return carry

        lax.fori_loop(0, _CHUNKS // 4, step, 0)
        wait_c(0)
        wait_c(1)

    return k(pids, cids, ce, p0t, p1t, c0t, c1t)


def kernel(pos_ids_0, pos_ids_1, cat_ids_0, cat_ids_1, continuous_feature,
           pos_table_0, pos_table_1, cat_table_0, cat_table_1, W_cont, b_cont):
    pids = jnp.stack([pos_ids_0.reshape(_N), pos_ids_1.reshape(_N)]
                     ).astype(jnp.int32).reshape(2, _NBLK, 128)
    cids = jnp.stack([cat_ids_0.reshape(_N), cat_ids_1.reshape(_N)]
                     ).astype(jnp.int32).reshape(2, _NBLK, 128)
    ce = _cont_proj(continuous_feature.reshape(_N, 16), W_cont,
                    b_cont.reshape(1, 32))
    out = _sc_assemble(pids, cids, ce,
                       pos_table_0, pos_table_1, cat_table_0, cat_table_1)
    return out.reshape(_B, _S, 128)
